# baseline (speedup 1.0000x reference)
import functools

import jax
import jax.numpy as jnp
from jax import lax
from jax.experimental import pallas as pl
from jax.experimental.pallas import tpu as pltpu
from jax.experimental.pallas import tpu_sc as plsc

NC, NS = 2, 16
NW = NC * NS
_INTERPRET = False


def _pack_cols(t):
    half = t.shape[-1] // 2
    lo = lax.bitcast_convert_type(t[:, :half].astype(jnp.bfloat16),
                                  jnp.uint16).astype(jnp.uint32)
    hi = lax.bitcast_convert_type(t[:, half:].astype(jnp.bfloat16),
                                  jnp.uint16).astype(jnp.uint32)
    return lax.bitcast_convert_type(lo | (hi << 16), jnp.int32)


def _unpack_cols(p):
    u = lax.bitcast_convert_type(p, jnp.uint32)
    lo = lax.bitcast_convert_type((u & 0xFFFF).astype(jnp.uint16),
                                  jnp.bfloat16).astype(jnp.float32)
    hi = lax.bitcast_convert_type((u >> 16).astype(jnp.uint16),
                                  jnp.bfloat16).astype(jnp.float32)
    return lo, hi


def _table_body(x_ref, w_ref, b_ref, t_ref):
    t_ref[0] = _pack_cols(
        jnp.dot(x_ref[...], w_ref[0], preferred_element_type=jnp.float32)
        + b_ref[0])


def _make_table(x, wab, bab, BN):
    N, D = x.shape
    return pl.pallas_call(
        _table_body,
        grid=(2, N // BN),
        in_specs=[
            pl.BlockSpec((BN, D), lambda h, j: (j, 0)),
            pl.BlockSpec((1, D, D), lambda h, j: (h, 0, 0)),
            pl.BlockSpec((1, 1, D), lambda h, j: (h, 0, 0)),
        ],
        out_specs=pl.BlockSpec((1, BN, D // 2), lambda h, j: (h, j, 0)),
        out_shape=jax.ShapeDtypeStruct((2, N, D // 2), jnp.int32),
        interpret=_INTERPRET,
    )(x, wab, bab)


def _edge_body(ga_ref, gb_ref, hb_ref, we_ref, we2_ref, be2_ref, out_ref):
    hb = hb_ref[...]
    ga_lo, ga_hi = _unpack_cols(ga_ref[0])
    gb_lo, gb_hi = _unpack_cols(gb_ref[0])
    g = jnp.concatenate([ga_lo + gb_lo, ga_hi + gb_hi], axis=1)
    x = g + jnp.dot(hb, we_ref[...], preferred_element_type=jnp.float32)
    x = jnp.maximum(x, 0.0)
    out_ref[...] = hb + jnp.dot(x, we2_ref[...],
                                preferred_element_type=jnp.float32) + be2_ref[...]


def _edge_mlp(G, hb, we, we2, be2, BE):
    E, D = hb.shape
    return pl.pallas_call(
        _edge_body,
        grid=(E // BE,),
        in_specs=[
            pl.BlockSpec((1, BE, D // 2), lambda j: (0, j, 0)),
            pl.BlockSpec((1, BE, D // 2), lambda j: (1, j, 0)),
            pl.BlockSpec((BE, D), lambda j: (j, 0)),
            pl.BlockSpec((D, D), lambda j: (0, 0)),
            pl.BlockSpec((D, D), lambda j: (0, 0)),
            pl.BlockSpec((1, D), lambda j: (0, 0)),
        ],
        out_specs=pl.BlockSpec((BE, D), lambda j: (j, 0)),
        out_shape=jax.ShapeDtypeStruct((E, D), jnp.float32),
        interpret=_INTERPRET,
    )(G, G, hb, we, we2, be2)


def _node_table_body(x_ref, agga_ref, aggb_ref, wn1x_ref, wn1a_ref, bn1_ref,
                     wn2_ref, bn2_ref, wab_ref, bab_ref, xo_ref, t_ref):
    xm = x_ref[...]
    agg = agga_ref[...] + aggb_ref[...]
    m = jnp.maximum(
        jnp.dot(xm, wn1x_ref[...], preferred_element_type=jnp.float32)
        + jnp.dot(agg, wn1a_ref[...], preferred_element_type=jnp.float32)
        + bn1_ref[...], 0.0)
    xn = xm + jnp.dot(m, wn2_ref[...],
                      preferred_element_type=jnp.float32) + bn2_ref[...]
    xo_ref[...] = xn
    t_ref[0] = _pack_cols(jnp.dot(xn, wab_ref[0],
                                  preferred_element_type=jnp.float32) + bab_ref[0])
    t_ref[1] = _pack_cols(jnp.dot(xn, wab_ref[1],
                                  preferred_element_type=jnp.float32) + bab_ref[1])


def _node_mlp_and_table(x, agga, aggb, wn1x, wn1a, bn1, wn2, bn2, wab, bab, BN):
    N, D = x.shape
    wspec = pl.BlockSpec((D, D), lambda j: (0, 0))
    bspec = pl.BlockSpec((1, D), lambda j: (0, 0))
    return pl.pallas_call(
        _node_table_body,
        grid=(N // BN,),
        in_specs=[
            pl.BlockSpec((BN, D), lambda j: (j, 0)),
            pl.BlockSpec((BN, D), lambda j: (j, 0)),
            pl.BlockSpec((BN, D), lambda j: (j, 0)),
            wspec, wspec, bspec, wspec, bspec,
            pl.BlockSpec((2, D, D), lambda j: (0, 0, 0)),
            pl.BlockSpec((2, D), lambda j: (0, 0)),
        ],
        out_specs=[
            pl.BlockSpec((BN, D), lambda j: (j, 0)),
            pl.BlockSpec((2, BN, D // 2), lambda j: (0, j, 0)),
        ],
        out_shape=[
            jax.ShapeDtypeStruct((N, D), jnp.float32),
            jax.ShapeDtypeStruct((2, N, D // 2), jnp.int32),
        ],
        interpret=_INTERPRET,
    )(x, agga, aggb, wn1x, wn1a, bn1, wn2, bn2, wab, bab)


def _node_body(x_ref, agga_ref, aggb_ref, wn1x_ref, wn1a_ref, bn1_ref,
               wn2_ref, bn2_ref, xo_ref):
    xm = x_ref[...]
    agg = agga_ref[...] + aggb_ref[...]
    m = jnp.maximum(
        jnp.dot(xm, wn1x_ref[...], preferred_element_type=jnp.float32)
        + jnp.dot(agg, wn1a_ref[...], preferred_element_type=jnp.float32)
        + bn1_ref[...], 0.0)
    xo_ref[...] = xm + jnp.dot(m, wn2_ref[...],
                               preferred_element_type=jnp.float32) + bn2_ref[...]


def _node_mlp(x, agga, aggb, wn1x, wn1a, bn1, wn2, bn2, BN):
    N, D = x.shape
    wspec = pl.BlockSpec((D, D), lambda j: (0, 0))
    bspec = pl.BlockSpec((1, D), lambda j: (0, 0))
    return pl.pallas_call(
        _node_body,
        grid=(N // BN,),
        in_specs=[
            pl.BlockSpec((BN, D), lambda j: (j, 0)),
            pl.BlockSpec((BN, D), lambda j: (j, 0)),
            pl.BlockSpec((BN, D), lambda j: (j, 0)),
            wspec, wspec, bspec, wspec, bspec,
        ],
        out_specs=pl.BlockSpec((BN, D), lambda j: (j, 0)),
        out_shape=jax.ShapeDtypeStruct((N, D), jnp.float32),
        interpret=_INTERPRET,
    )(x, agga, aggb, wn1x, wn1a, bn1, wn2, bn2)


def _gather_rows(tbl, idx):
    TOT = idx.shape[0]
    D = tbl.shape[1]
    per_w = TOT // NW
    CH = 80
    n_ch = per_w // CH
    n2 = n_ch // 2
    mesh = plsc.VectorSubcoreMesh(core_axis_name="c", subcore_axis_name="s")

    @functools.partial(
        pl.kernel,
        out_type=jax.ShapeDtypeStruct((TOT, D), jnp.int32),
        mesh=mesh,
        scratch_types=[
            pltpu.VMEM((per_w,), jnp.int32),
            pltpu.VMEM((2, CH, D), jnp.int32),
            pltpu.SemaphoreType.DMA,
            pltpu.SemaphoreType.DMA,
        ],
    )
    def gk(tbl_hbm, idx_hbm, out_hbm, idx_w, rows_v, sem0, sem1):
        wid = lax.axis_index("s") * NC + lax.axis_index("c")
        base = wid * per_w
        pltpu.sync_copy(idx_hbm.at[pl.ds(base, per_w)], idx_w)

        def start(ci, b, sem):
            pltpu.async_copy(tbl_hbm.at[idx_w.at[pl.ds(ci * CH, CH)]],
                             rows_v.at[b], sem)

        def wait(b, sem):
            pltpu.make_async_copy(tbl_hbm.at[idx_w.at[pl.ds(0, CH)]],
                                  rows_v.at[b], sem).wait()

        def out(ci, b):
            pltpu.sync_copy(rows_v.at[b], out_hbm.at[pl.ds(base + ci * CH, CH)])

        start(0, 0, sem0)

        def body(g, carry):
            c0 = 2 * g
            start(c0 + 1, 1, sem1)
            wait(0, sem0)
            out(c0, 0)

            @pl.when(c0 + 2 < n_ch)
            def _():
                start(c0 + 2, 0, sem0)

            wait(1, sem1)
            out(c0 + 1, 1)
            return carry

        lax.fori_loop(0, n2, body, 0)
        if n_ch % 2:
            wait(0, sem0)
            out(n_ch - 1, 0)

    return gk(tbl, idx)


def _segment_sum(hb, dst, zrows, N):
    E, D = hb.shape
    COLS = D // NC
    R0 = ((N + NS - 1) // NS + 7) // 8 * 8
    LAST = N - (NS - 1) * R0
    EPT = E // NS
    CH = 80
    n_ch = EPT // CH
    mesh = plsc.VectorSubcoreMesh(core_axis_name="c", subcore_axis_name="s")

    n2 = n_ch // 2

    @functools.partial(
        pl.kernel,
        out_type=jax.ShapeDtypeStruct((N, D), jnp.float32),
        mesh=mesh,
        scratch_types=[
            pltpu.VMEM((n_ch, CH), jnp.int32),
            pltpu.VMEM((2, CH, COLS), jnp.float32),
            pltpu.VMEM_SHARED((N, COLS), jnp.float32),
            pltpu.SemaphoreType.DMA,
            pltpu.SemaphoreType.DMA,
        ],
    )
    def sk(hb_hbm, dst3_hbm, z_hbm, out_hbm, idx2, rows_v, acc, sem0, sem1):
        c = lax.axis_index("c")
        s = lax.axis_index("s")
        pltpu.sync_copy(dst3_hbm.at[s], idx2)

        @pl.when(s < NS - 1)
        def _():
            pltpu.sync_copy(z_hbm, acc.at[pl.ds(s * R0, R0)])

        @pl.when(s == NS - 1)
        def _():
            pltpu.sync_copy(z_hbm.at[pl.ds(0, LAST)],
                            acc.at[pl.ds((NS - 1) * R0, LAST)])

        plsc.subcore_barrier()

        def start(ci, b, sem):
            e0 = s * EPT + ci * CH
            pltpu.async_copy(hb_hbm.at[pl.ds(e0, CH), pl.ds(c * COLS, COLS)],
                             rows_v.at[b], sem)

        def wait(b, sem):
            pltpu.make_async_copy(
                hb_hbm.at[pl.ds(0, CH), pl.ds(c * COLS, COLS)],
                rows_v.at[b], sem).wait()

        def add(ci, b):
            pltpu.sync_copy(rows_v.at[b], acc.at[idx2.at[ci]], add=True)

        start(0, 0, sem0)

        def body(g, carry):
            c0 = 2 * g
            start(c0 + 1, 1, sem1)
            wait(0, sem0)
            add(c0, 0)

            @pl.when(c0 + 2 < n_ch)
            def _():
                start(c0 + 2, 0, sem0)

            wait(1, sem1)
            add(c0 + 1, 1)
            return carry

        lax.fori_loop(0, n2, body, 0)
        if n_ch % 2:
            wait(0, sem0)
            add(n_ch - 1, 0)
        plsc.subcore_barrier()

        @pl.when(s < NS - 1)
        def _():
            pltpu.sync_copy(acc.at[pl.ds(s * R0, R0)],
                            out_hbm.at[pl.ds(s * R0, R0), pl.ds(c * COLS, COLS)])

        @pl.when(s == NS - 1)
        def _():
            pltpu.sync_copy(acc.at[pl.ds((NS - 1) * R0, LAST)],
                            out_hbm.at[pl.ds((NS - 1) * R0, LAST),
                                       pl.ds(c * COLS, COLS)])

    return sk(hb, dst.reshape(NS, n_ch, CH), zrows)


def kernel(h_atm, edge_index_G, h_bnd, We1, be1, We2, be2, Wn1, bn1, Wn2, bn2):
    N, D = h_atm.shape
    E = h_bnd.shape[0]
    L = We1.shape[0]
    BN = 2000 if N % 2000 == 0 else N
    BE = 1280

    src = edge_index_G[0]
    dst = edge_index_G[1]
    BE = BE if E % BE == 0 else E
    E2 = (E // 2) // BE * BE
    parts = [(0, E2), (E2, E - E2)] if 0 < E2 < E else [(0, E)]
    NP = len(parts)

    Wa = We1[:, :D, :]
    Wb = We1[:, D:2 * D, :]
    We = We1[:, 2 * D:, :]
    Wab = jnp.stack([Wa, Wb], axis=1)
    Bab = jnp.stack([be1, jnp.zeros_like(be1)], axis=1)
    Wn1x = Wn1[:, :D, :]
    Wn1a = Wn1[:, D:, :]
    R0 = ((N + NS - 1) // NS + 7) // 8 * 8
    zrows = jnp.zeros((R0, D // NC), dtype=jnp.float32)

    idx_p = [jnp.concatenate([src[o:o + n], dst[o:o + n] + N]) for o, n in parts]
    dst_p = [dst[o:o + n] for o, n in parts]
    hbs = [h_bnd[o:o + n] for o, n in parts]

    T = _make_table(h_atm, Wab[0], Bab[0].reshape(2, 1, D), BN)
    for i in range(L):
        be2_i = be2[i].reshape(1, D)
        Gs = [_gather_rows(T.reshape(2 * N, D // 2), idx_p[p])
              .reshape(2, parts[p][1], D // 2) for p in range(NP)]
        aggs = []
        for p in range(NP):
            hbs[p] = _edge_mlp(Gs[p], hbs[p], We[i], We2[i], be2_i, BE)
            aggs.append(_segment_sum(hbs[p], dst_p[p], zrows, N))
        if NP == 1:
            aggs.append(jnp.zeros_like(aggs[0]))
        bn1_i = bn1[i].reshape(1, D)
        bn2_i = bn2[i].reshape(1, D)
        if i + 1 < L:
            h_atm, T = _node_mlp_and_table(h_atm, aggs[0], aggs[1], Wn1x[i],
                                           Wn1a[i], bn1_i, Wn2[i], bn2_i,
                                           Wab[i + 1], Bab[i + 1], BN)
        else:
            h_atm = _node_mlp(h_atm, aggs[0], aggs[1], Wn1x[i], Wn1a[i],
                              bn1_i, Wn2[i], bn2_i, BN)
    return (h_atm, jnp.concatenate(hbs, axis=0))


# speedup vs baseline: 1.0997x; 1.0997x over previous
import functools

import jax
import jax.numpy as jnp
from jax import lax
from jax.experimental import pallas as pl
from jax.experimental.pallas import tpu as pltpu
from jax.experimental.pallas import tpu_sc as plsc

NC, NS = 2, 16
NW = NC * NS
_INTERPRET = False


def _pack_cols(t):
    half = t.shape[-1] // 2
    lo = lax.bitcast_convert_type(t[:, :half].astype(jnp.bfloat16),
                                  jnp.uint16).astype(jnp.uint32)
    hi = lax.bitcast_convert_type(t[:, half:].astype(jnp.bfloat16),
                                  jnp.uint16).astype(jnp.uint32)
    return lax.bitcast_convert_type(lo | (hi << 16), jnp.int32)


def _unpack_cols(p):
    u = lax.bitcast_convert_type(p, jnp.uint32)
    lo = lax.bitcast_convert_type((u & 0xFFFF).astype(jnp.uint16),
                                  jnp.bfloat16).astype(jnp.float32)
    hi = lax.bitcast_convert_type((u >> 16).astype(jnp.uint16),
                                  jnp.bfloat16).astype(jnp.float32)
    return lo, hi


def _table_body(x_ref, w_ref, b_ref, t_ref):
    t_ref[0] = _pack_cols(
        jnp.dot(x_ref[...], w_ref[0], preferred_element_type=jnp.float32)
        + b_ref[0])


def _make_table(x, wab, bab, BN):
    N, D = x.shape
    return pl.pallas_call(
        _table_body,
        grid=(2, N // BN),
        in_specs=[
            pl.BlockSpec((BN, D), lambda h, j: (j, 0)),
            pl.BlockSpec((1, D, D), lambda h, j: (h, 0, 0)),
            pl.BlockSpec((1, 1, D), lambda h, j: (h, 0, 0)),
        ],
        out_specs=pl.BlockSpec((1, BN, D // 2), lambda h, j: (h, j, 0)),
        out_shape=jax.ShapeDtypeStruct((2, N, D // 2), jnp.int32),
        interpret=_INTERPRET,
    )(x, wab, bab)


def _edge_body(ga_ref, gb_ref, hb_ref, we_ref, we2_ref, be2_ref, out_ref):
    hb = hb_ref[...]
    ga_lo, ga_hi = _unpack_cols(ga_ref[0])
    gb_lo, gb_hi = _unpack_cols(gb_ref[0])
    g = jnp.concatenate([ga_lo + gb_lo, ga_hi + gb_hi], axis=1)
    x = g + jnp.dot(hb, we_ref[...], preferred_element_type=jnp.float32)
    x = jnp.maximum(x, 0.0)
    out_ref[...] = hb + jnp.dot(x, we2_ref[...],
                                preferred_element_type=jnp.float32) + be2_ref[...]


def _edge_mlp(G, hb, we, we2, be2, BE):
    E, D = hb.shape
    return pl.pallas_call(
        _edge_body,
        grid=(E // BE,),
        in_specs=[
            pl.BlockSpec((1, BE, D // 2), lambda j: (0, j, 0)),
            pl.BlockSpec((1, BE, D // 2), lambda j: (1, j, 0)),
            pl.BlockSpec((BE, D), lambda j: (j, 0)),
            pl.BlockSpec((D, D), lambda j: (0, 0)),
            pl.BlockSpec((D, D), lambda j: (0, 0)),
            pl.BlockSpec((1, D), lambda j: (0, 0)),
        ],
        out_specs=pl.BlockSpec((BE, D), lambda j: (j, 0)),
        out_shape=jax.ShapeDtypeStruct((E, D), jnp.float32),
        interpret=_INTERPRET,
    )(G, G, hb, we, we2, be2)


def _node_table_body(x_ref, agg_ref, wn1x_ref, wn1a_ref, bn1_ref,
                     wn2_ref, bn2_ref, wab_ref, bab_ref, xo_ref, t_ref):
    xm = x_ref[...]
    m = jnp.maximum(
        jnp.dot(xm, wn1x_ref[...], preferred_element_type=jnp.float32)
        + jnp.dot(agg_ref[...], wn1a_ref[...], preferred_element_type=jnp.float32)
        + bn1_ref[...], 0.0)
    xn = xm + jnp.dot(m, wn2_ref[...],
                      preferred_element_type=jnp.float32) + bn2_ref[...]
    xo_ref[...] = xn
    t_ref[0] = _pack_cols(jnp.dot(xn, wab_ref[0],
                                  preferred_element_type=jnp.float32) + bab_ref[0])
    t_ref[1] = _pack_cols(jnp.dot(xn, wab_ref[1],
                                  preferred_element_type=jnp.float32) + bab_ref[1])


def _node_mlp_and_table(x, agg, wn1x, wn1a, bn1, wn2, bn2, wab, bab, BN):
    N, D = x.shape
    wspec = pl.BlockSpec((D, D), lambda j: (0, 0))
    bspec = pl.BlockSpec((1, D), lambda j: (0, 0))
    return pl.pallas_call(
        _node_table_body,
        grid=(N // BN,),
        in_specs=[
            pl.BlockSpec((BN, D), lambda j: (j, 0)),
            pl.BlockSpec((BN, D), lambda j: (j, 0)),
            wspec, wspec, bspec, wspec, bspec,
            pl.BlockSpec((2, D, D), lambda j: (0, 0, 0)),
            pl.BlockSpec((2, D), lambda j: (0, 0)),
        ],
        out_specs=[
            pl.BlockSpec((BN, D), lambda j: (j, 0)),
            pl.BlockSpec((2, BN, D // 2), lambda j: (0, j, 0)),
        ],
        out_shape=[
            jax.ShapeDtypeStruct((N, D), jnp.float32),
            jax.ShapeDtypeStruct((2, N, D // 2), jnp.int32),
        ],
        interpret=_INTERPRET,
    )(x, agg, wn1x, wn1a, bn1, wn2, bn2, wab, bab)


def _node_body(x_ref, agg_ref, wn1x_ref, wn1a_ref, bn1_ref,
               wn2_ref, bn2_ref, xo_ref):
    xm = x_ref[...]
    m = jnp.maximum(
        jnp.dot(xm, wn1x_ref[...], preferred_element_type=jnp.float32)
        + jnp.dot(agg_ref[...], wn1a_ref[...], preferred_element_type=jnp.float32)
        + bn1_ref[...], 0.0)
    xo_ref[...] = xm + jnp.dot(m, wn2_ref[...],
                               preferred_element_type=jnp.float32) + bn2_ref[...]


def _node_mlp(x, agg, wn1x, wn1a, bn1, wn2, bn2, BN):
    N, D = x.shape
    wspec = pl.BlockSpec((D, D), lambda j: (0, 0))
    bspec = pl.BlockSpec((1, D), lambda j: (0, 0))
    return pl.pallas_call(
        _node_body,
        grid=(N // BN,),
        in_specs=[
            pl.BlockSpec((BN, D), lambda j: (j, 0)),
            pl.BlockSpec((BN, D), lambda j: (j, 0)),
            wspec, wspec, bspec, wspec, bspec,
        ],
        out_specs=pl.BlockSpec((BN, D), lambda j: (j, 0)),
        out_shape=jax.ShapeDtypeStruct((N, D), jnp.float32),
        interpret=_INTERPRET,
    )(x, agg, wn1x, wn1a, bn1, wn2, bn2)


def _gather_rows(tbl, idx):
    TOT = idx.shape[0]
    D = tbl.shape[1]
    per_w = TOT // NW
    CH = 80
    n_ch = per_w // CH
    n2 = n_ch // 2
    mesh = plsc.VectorSubcoreMesh(core_axis_name="c", subcore_axis_name="s")

    GR = 5 if n_ch % 5 == 0 else 1
    RG = GR * CH
    n_g = n_ch // GR
    pairs, rem = n_g // 2, n_g % 2

    @functools.partial(
        pl.kernel,
        out_type=jax.ShapeDtypeStruct((TOT, D), jnp.int32),
        mesh=mesh,
        scratch_types=[
            pltpu.VMEM((per_w,), jnp.int32),
            pltpu.VMEM((2, RG, D), jnp.int32),
            pltpu.SemaphoreType.DMA,
            pltpu.SemaphoreType.DMA,
            pltpu.SemaphoreType.DMA,
            pltpu.SemaphoreType.DMA,
        ],
    )
    def gk(tbl_hbm, idx_hbm, out_hbm, idx_w, rows_v, sg0, sg1, so0, so1):
        wid = lax.axis_index("s") * NC + lax.axis_index("c")
        base = wid * per_w
        sg = (sg0, sg1)
        so = (so0, so1)
        pltpu.sync_copy(idx_hbm.at[pl.ds(base, per_w)], idx_w)

        def start_group(G, b):
            for k in range(GR):
                pltpu.async_copy(
                    tbl_hbm.at[idx_w.at[pl.ds(G * RG + k * CH, CH)]],
                    rows_v.at[b, pl.ds(k * CH, CH)], sg[b])

        def wait_group(b):
            pltpu.make_async_copy(out_hbm.at[pl.ds(base, RG)], rows_v.at[b],
                                  sg[b]).wait()

        def start_out(G, b):
            pltpu.async_copy(rows_v.at[b],
                             out_hbm.at[pl.ds(base + G * RG, RG)], so[b])

        def wait_out(b):
            pltpu.make_async_copy(out_hbm.at[pl.ds(base, RG)], rows_v.at[b],
                                  so[b]).wait()

        start_group(0, 0)

        def body(g2, carry):
            for pb in (0, 1):
                G = 2 * g2 + pb
                q = 1 - pb

                @pl.when(G + 1 < n_g)
                def _():
                    @pl.when(G >= 1)
                    def _():
                        wait_out(q)
                    start_group(G + 1, q)

                wait_group(pb)
                start_out(G, pb)
            return carry

        lax.fori_loop(0, pairs, body, 0)
        if rem:
            Gt = n_g - 1
            wait_group(Gt % 2)
            start_out(Gt, Gt % 2)
        if n_g >= 2:
            wait_out((n_g - 2) % 2)
        wait_out((n_g - 1) % 2)

    return gk(tbl, idx)


def _segment_sum(hb, dst, zrows, N):
    E, D = hb.shape
    COLS = D // NC
    R0 = ((N + NS - 1) // NS + 7) // 8 * 8
    LAST = N - (NS - 1) * R0
    EPT = E // NS
    CH = 80
    n_ch = EPT // CH
    mesh = plsc.VectorSubcoreMesh(core_axis_name="c", subcore_axis_name="s")

    GR = 1
    RG = GR * CH
    n_g = n_ch // GR
    pairs, rem = n_g // 2, n_g % 2

    @functools.partial(
        pl.kernel,
        out_type=jax.ShapeDtypeStruct((N, D), jnp.float32),
        mesh=mesh,
        scratch_types=[
            pltpu.VMEM((n_ch, CH), jnp.int32),
            pltpu.VMEM((2, RG, COLS), jnp.float32),
            pltpu.VMEM_SHARED((N, COLS), jnp.float32),
            pltpu.SemaphoreType.DMA,
            pltpu.SemaphoreType.DMA,
            pltpu.SemaphoreType.DMA,
            pltpu.SemaphoreType.DMA,
        ],
    )
    def sk(hb_hbm, dst3_hbm, z_hbm, out_hbm, idx2, rows_v, acc,
           sr0, sr1, sa0, sa1):
        c = lax.axis_index("c")
        s = lax.axis_index("s")
        sr = (sr0, sr1)
        sa = (sa0, sa1)
        pltpu.sync_copy(dst3_hbm.at[s], idx2)

        @pl.when(s < NS - 1)
        def _():
            pltpu.sync_copy(z_hbm, acc.at[pl.ds(s * R0, R0)])

        @pl.when(s == NS - 1)
        def _():
            pltpu.sync_copy(z_hbm.at[pl.ds(0, LAST)],
                            acc.at[pl.ds((NS - 1) * R0, LAST)])

        plsc.subcore_barrier()

        def start_read(G, b):
            e0 = s * EPT + G * RG
            pltpu.async_copy(hb_hbm.at[pl.ds(e0, RG), pl.ds(c * COLS, COLS)],
                             rows_v.at[b], sr[b])

        def wait_read(b):
            pltpu.make_async_copy(
                hb_hbm.at[pl.ds(0, RG), pl.ds(c * COLS, COLS)],
                rows_v.at[b], sr[b]).wait()

        def start_adds(G, b):
            for k in range(GR):
                pltpu.async_copy(rows_v.at[b, pl.ds(k * CH, CH)],
                                 acc.at[idx2.at[G * GR + k]], sa[b], add=True)

        def wait_adds(b):
            pltpu.make_async_copy(
                hb_hbm.at[pl.ds(0, RG), pl.ds(c * COLS, COLS)],
                rows_v.at[b], sa[b]).wait()

        start_read(0, 0)

        def body(g2, carry):
            for pb in (0, 1):
                G = 2 * g2 + pb
                q = 1 - pb

                @pl.when(G + 1 < n_g)
                def _():
                    @pl.when(G >= 1)
                    def _():
                        wait_adds(q)
                    start_read(G + 1, q)

                wait_read(pb)
                start_adds(G, pb)
            return carry

        lax.fori_loop(0, pairs, body, 0)
        if rem:
            Gt = n_g - 1
            wait_read(Gt % 2)
            start_adds(Gt, Gt % 2)
        if n_g >= 2:
            wait_adds((n_g - 2) % 2)
        wait_adds((n_g - 1) % 2)
        plsc.subcore_barrier()

        @pl.when(s < NS - 1)
        def _():
            pltpu.sync_copy(acc.at[pl.ds(s * R0, R0)],
                            out_hbm.at[pl.ds(s * R0, R0), pl.ds(c * COLS, COLS)])

        @pl.when(s == NS - 1)
        def _():
            pltpu.sync_copy(acc.at[pl.ds((NS - 1) * R0, LAST)],
                            out_hbm.at[pl.ds((NS - 1) * R0, LAST),
                                       pl.ds(c * COLS, COLS)])

    return sk(hb, dst.reshape(NS, n_ch, CH), zrows)


def kernel(h_atm, edge_index_G, h_bnd, We1, be1, We2, be2, Wn1, bn1, Wn2, bn2):
    N, D = h_atm.shape
    E = h_bnd.shape[0]
    L = We1.shape[0]
    BN = 2000 if N % 2000 == 0 else N
    BE = 2000

    src = edge_index_G[0]
    dst = edge_index_G[1]
    BE = BE if E % BE == 0 else E
    parts = [(0, E)]
    NP = len(parts)

    Wa = We1[:, :D, :]
    Wb = We1[:, D:2 * D, :]
    We = We1[:, 2 * D:, :]
    Wab = jnp.stack([Wa, Wb], axis=1)
    Bab = jnp.stack([be1, jnp.zeros_like(be1)], axis=1)
    Wn1x = Wn1[:, :D, :]
    Wn1a = Wn1[:, D:, :]
    R0 = ((N + NS - 1) // NS + 7) // 8 * 8
    zrows = jnp.zeros((R0, D // NC), dtype=jnp.float32)

    idx_p = [jnp.concatenate([src[o:o + n], dst[o:o + n] + N]) for o, n in parts]
    dst_p = [dst[o:o + n] for o, n in parts]
    hbs = [h_bnd[o:o + n] for o, n in parts]

    T = _make_table(h_atm, Wab[0], Bab[0].reshape(2, 1, D), BN)
    for i in range(L):
        be2_i = be2[i].reshape(1, D)
        Gs = [_gather_rows(T.reshape(2 * N, D // 2), idx_p[p])
              .reshape(2, parts[p][1], D // 2) for p in range(NP)]
        aggs = []
        for p in range(NP):
            hbs[p] = _edge_mlp(Gs[p], hbs[p], We[i], We2[i], be2_i, BE)
            aggs.append(_segment_sum(hbs[p], dst_p[p], zrows, N))
        agg = aggs[0] if NP == 1 else sum(aggs)
        bn1_i = bn1[i].reshape(1, D)
        bn2_i = bn2[i].reshape(1, D)
        if i + 1 < L:
            h_atm, T = _node_mlp_and_table(h_atm, agg, Wn1x[i],
                                           Wn1a[i], bn1_i, Wn2[i], bn2_i,
                                           Wab[i + 1], Bab[i + 1], BN)
        else:
            h_atm = _node_mlp(h_atm, agg, Wn1x[i], Wn1a[i],
                              bn1_i, Wn2[i], bn2_i, BN)
    return (h_atm, jnp.concatenate(hbs, axis=0))


# speedup vs baseline: 1.1270x; 1.0249x over previous
import functools

import jax
import jax.numpy as jnp
from jax import lax
from jax.experimental import pallas as pl
from jax.experimental.pallas import tpu as pltpu
from jax.experimental.pallas import tpu_sc as plsc

NC, NS = 2, 16
NW = NC * NS
_INTERPRET = False


def _pack_cols(t):
    half = t.shape[-1] // 2
    lo = lax.bitcast_convert_type(t[:, :half].astype(jnp.bfloat16),
                                  jnp.uint16).astype(jnp.uint32)
    hi = lax.bitcast_convert_type(t[:, half:].astype(jnp.bfloat16),
                                  jnp.uint16).astype(jnp.uint32)
    return lax.bitcast_convert_type(lo | (hi << 16), jnp.int32)


def _unpack_cols(p):
    u = lax.bitcast_convert_type(p, jnp.uint32)
    lo = lax.bitcast_convert_type((u & 0xFFFF).astype(jnp.uint16),
                                  jnp.bfloat16).astype(jnp.float32)
    hi = lax.bitcast_convert_type((u >> 16).astype(jnp.uint16),
                                  jnp.bfloat16).astype(jnp.float32)
    return lo, hi


def _table_body(x_ref, w_ref, b_ref, t_ref):
    t_ref[0] = _pack_cols(
        jnp.dot(x_ref[...], w_ref[0], preferred_element_type=jnp.float32)
        + b_ref[0])


def _make_table(x, wab, bab, BN):
    N, D = x.shape
    return pl.pallas_call(
        _table_body,
        grid=(2, N // BN),
        in_specs=[
            pl.BlockSpec((BN, D), lambda h, j: (j, 0)),
            pl.BlockSpec((1, D, D), lambda h, j: (h, 0, 0)),
            pl.BlockSpec((1, 1, D), lambda h, j: (h, 0, 0)),
        ],
        out_specs=pl.BlockSpec((1, BN, D // 2), lambda h, j: (h, j, 0)),
        out_shape=jax.ShapeDtypeStruct((2, N, D // 2), jnp.int32),
        interpret=_INTERPRET,
    )(x, wab, bab)


def _edge_body(ga_ref, gb_ref, hb_ref, we_ref, we2_ref, be2_ref, out_ref):
    hb = hb_ref[...]
    ga_lo, ga_hi = _unpack_cols(ga_ref[0])
    gb_lo, gb_hi = _unpack_cols(gb_ref[0])
    g = jnp.concatenate([ga_lo + gb_lo, ga_hi + gb_hi], axis=1)
    x = g + jnp.dot(hb, we_ref[...], preferred_element_type=jnp.float32)
    x = jnp.maximum(x, 0.0)
    out_ref[...] = hb + jnp.dot(x, we2_ref[...],
                                preferred_element_type=jnp.float32) + be2_ref[...]


def _edge_mlp(G, hb, we, we2, be2, BE):
    E, D = hb.shape
    return pl.pallas_call(
        _edge_body,
        grid=(E // BE,),
        in_specs=[
            pl.BlockSpec((1, BE, D // 2), lambda j: (0, j, 0)),
            pl.BlockSpec((1, BE, D // 2), lambda j: (1, j, 0)),
            pl.BlockSpec((BE, D), lambda j: (j, 0)),
            pl.BlockSpec((D, D), lambda j: (0, 0)),
            pl.BlockSpec((D, D), lambda j: (0, 0)),
            pl.BlockSpec((1, D), lambda j: (0, 0)),
        ],
        out_specs=pl.BlockSpec((BE, D), lambda j: (j, 0)),
        out_shape=jax.ShapeDtypeStruct((E, D), jnp.float32),
        interpret=_INTERPRET,
    )(G, G, hb, we, we2, be2)


def _node_table_body(x_ref, agg_ref, wn1x_ref, wn1a_ref, bn1_ref,
                     wn2_ref, bn2_ref, wab_ref, bab_ref, xo_ref, t_ref):
    xm = x_ref[...]
    m = jnp.maximum(
        jnp.dot(xm, wn1x_ref[...], preferred_element_type=jnp.float32)
        + jnp.dot(agg_ref[...], wn1a_ref[...], preferred_element_type=jnp.float32)
        + bn1_ref[...], 0.0)
    xn = xm + jnp.dot(m, wn2_ref[...],
                      preferred_element_type=jnp.float32) + bn2_ref[...]
    xo_ref[...] = xn
    t_ref[0] = _pack_cols(jnp.dot(xn, wab_ref[0],
                                  preferred_element_type=jnp.float32) + bab_ref[0])
    t_ref[1] = _pack_cols(jnp.dot(xn, wab_ref[1],
                                  preferred_element_type=jnp.float32) + bab_ref[1])


def _node_mlp_and_table(x, agg, wn1x, wn1a, bn1, wn2, bn2, wab, bab, BN):
    N, D = x.shape
    wspec = pl.BlockSpec((D, D), lambda j: (0, 0))
    bspec = pl.BlockSpec((1, D), lambda j: (0, 0))
    return pl.pallas_call(
        _node_table_body,
        grid=(N // BN,),
        in_specs=[
            pl.BlockSpec((BN, D), lambda j: (j, 0)),
            pl.BlockSpec((BN, D), lambda j: (j, 0)),
            wspec, wspec, bspec, wspec, bspec,
            pl.BlockSpec((2, D, D), lambda j: (0, 0, 0)),
            pl.BlockSpec((2, D), lambda j: (0, 0)),
        ],
        out_specs=[
            pl.BlockSpec((BN, D), lambda j: (j, 0)),
            pl.BlockSpec((2, BN, D // 2), lambda j: (0, j, 0)),
        ],
        out_shape=[
            jax.ShapeDtypeStruct((N, D), jnp.float32),
            jax.ShapeDtypeStruct((2, N, D // 2), jnp.int32),
        ],
        interpret=_INTERPRET,
    )(x, agg, wn1x, wn1a, bn1, wn2, bn2, wab, bab)


def _node_body(x_ref, agg_ref, wn1x_ref, wn1a_ref, bn1_ref,
               wn2_ref, bn2_ref, xo_ref):
    xm = x_ref[...]
    m = jnp.maximum(
        jnp.dot(xm, wn1x_ref[...], preferred_element_type=jnp.float32)
        + jnp.dot(agg_ref[...], wn1a_ref[...], preferred_element_type=jnp.float32)
        + bn1_ref[...], 0.0)
    xo_ref[...] = xm + jnp.dot(m, wn2_ref[...],
                               preferred_element_type=jnp.float32) + bn2_ref[...]


def _node_mlp(x, agg, wn1x, wn1a, bn1, wn2, bn2, BN):
    N, D = x.shape
    wspec = pl.BlockSpec((D, D), lambda j: (0, 0))
    bspec = pl.BlockSpec((1, D), lambda j: (0, 0))
    return pl.pallas_call(
        _node_body,
        grid=(N // BN,),
        in_specs=[
            pl.BlockSpec((BN, D), lambda j: (j, 0)),
            pl.BlockSpec((BN, D), lambda j: (j, 0)),
            wspec, wspec, bspec, wspec, bspec,
        ],
        out_specs=pl.BlockSpec((BN, D), lambda j: (j, 0)),
        out_shape=jax.ShapeDtypeStruct((N, D), jnp.float32),
        interpret=_INTERPRET,
    )(x, agg, wn1x, wn1a, bn1, wn2, bn2)


def _gather_rows(tbl, idx):
    TOT = idx.shape[0]
    D = tbl.shape[1]
    per_w = TOT // NW
    CH = 80
    n_ch = per_w // CH
    n2 = n_ch // 2
    mesh = plsc.VectorSubcoreMesh(core_axis_name="c", subcore_axis_name="s")

    GR = 5 if n_ch % 5 == 0 else 1
    RG = GR * CH
    n_g = n_ch // GR
    pairs, rem = n_g // 2, n_g % 2

    @functools.partial(
        pl.kernel,
        out_type=jax.ShapeDtypeStruct((TOT, D), jnp.int32),
        mesh=mesh,
        scratch_types=[
            pltpu.VMEM((per_w,), jnp.int32),
            pltpu.VMEM((2, RG, D), jnp.int32),
            pltpu.SemaphoreType.DMA,
            pltpu.SemaphoreType.DMA,
            pltpu.SemaphoreType.DMA,
            pltpu.SemaphoreType.DMA,
        ],
    )
    def gk(tbl_hbm, idx_hbm, out_hbm, idx_w, rows_v, sg0, sg1, so0, so1):
        wid = lax.axis_index("s") * NC + lax.axis_index("c")
        base = wid * per_w
        sg = (sg0, sg1)
        so = (so0, so1)
        pltpu.sync_copy(idx_hbm.at[pl.ds(base, per_w)], idx_w)

        def start_group(G, b):
            for k in range(GR):
                pltpu.async_copy(
                    tbl_hbm.at[idx_w.at[pl.ds(G * RG + k * CH, CH)]],
                    rows_v.at[b, pl.ds(k * CH, CH)], sg[b])

        def wait_group(b):
            pltpu.make_async_copy(out_hbm.at[pl.ds(base, RG)], rows_v.at[b],
                                  sg[b]).wait()

        def start_out(G, b):
            pltpu.async_copy(rows_v.at[b],
                             out_hbm.at[pl.ds(base + G * RG, RG)], so[b])

        def wait_out(b):
            pltpu.make_async_copy(out_hbm.at[pl.ds(base, RG)], rows_v.at[b],
                                  so[b]).wait()

        start_group(0, 0)

        def body(g2, carry):
            for pb in (0, 1):
                G = 2 * g2 + pb
                q = 1 - pb

                @pl.when(G + 1 < n_g)
                def _():
                    @pl.when(G >= 1)
                    def _():
                        wait_out(q)
                    start_group(G + 1, q)

                wait_group(pb)
                start_out(G, pb)
            return carry

        lax.fori_loop(0, pairs, body, 0)
        if rem:
            Gt = n_g - 1
            wait_group(Gt % 2)
            start_out(Gt, Gt % 2)
        if n_g >= 2:
            wait_out((n_g - 2) % 2)
        wait_out((n_g - 1) % 2)

    return gk(tbl, idx)


def _segment_sum(hb, dst, zrows, N):
    E, D = hb.shape
    COLS = D // NC
    R0 = ((N + NS - 1) // NS + 7) // 8 * 8
    LAST = N - (NS - 1) * R0
    EPT = E // NS
    CH = 80
    n_ch = EPT // CH
    mesh = plsc.VectorSubcoreMesh(core_axis_name="c", subcore_axis_name="s")

    GR = 1
    RG = GR * CH
    n_g = n_ch // GR
    pairs, rem = n_g // 2, n_g % 2

    @functools.partial(
        pl.kernel,
        out_type=jax.ShapeDtypeStruct((N, D), jnp.float32),
        mesh=mesh,
        scratch_types=[
            pltpu.VMEM((n_ch, CH), jnp.int32),
            pltpu.VMEM((2, RG, COLS), jnp.float32),
            pltpu.VMEM_SHARED((N, COLS), jnp.float32),
            pltpu.SemaphoreType.DMA,
            pltpu.SemaphoreType.DMA,
            pltpu.SemaphoreType.DMA,
            pltpu.SemaphoreType.DMA,
        ],
    )
    def sk(hb_hbm, dst3_hbm, z_hbm, out_hbm, idx2, rows_v, acc,
           sr0, sr1, sa0, sa1):
        c = lax.axis_index("c")
        s = lax.axis_index("s")
        sr = (sr0, sr1)
        sa = (sa0, sa1)
        pltpu.sync_copy(dst3_hbm.at[s], idx2)

        @pl.when(s < NS - 1)
        def _():
            pltpu.sync_copy(z_hbm, acc.at[pl.ds(s * R0, R0)])

        @pl.when(s == NS - 1)
        def _():
            pltpu.sync_copy(z_hbm.at[pl.ds(0, LAST)],
                            acc.at[pl.ds((NS - 1) * R0, LAST)])

        plsc.subcore_barrier()

        def start_read(G, b):
            e0 = s * EPT + G * RG
            pltpu.async_copy(hb_hbm.at[pl.ds(e0, RG), pl.ds(c * COLS, COLS)],
                             rows_v.at[b], sr[b])

        def wait_read(b):
            pltpu.make_async_copy(
                hb_hbm.at[pl.ds(0, RG), pl.ds(c * COLS, COLS)],
                rows_v.at[b], sr[b]).wait()

        def start_adds(G, b):
            for k in range(GR):
                pltpu.async_copy(rows_v.at[b, pl.ds(k * CH, CH)],
                                 acc.at[idx2.at[G * GR + k]], sa[b], add=True)

        def wait_adds(b):
            pltpu.make_async_copy(
                hb_hbm.at[pl.ds(0, RG), pl.ds(c * COLS, COLS)],
                rows_v.at[b], sa[b]).wait()

        start_read(0, 0)

        def body(g2, carry):
            for pb in (0, 1):
                G = 2 * g2 + pb
                q = 1 - pb

                @pl.when(G + 1 < n_g)
                def _():
                    @pl.when(G >= 1)
                    def _():
                        wait_adds(q)
                    start_read(G + 1, q)

                wait_read(pb)
                start_adds(G, pb)
            return carry

        lax.fori_loop(0, pairs, body, 0)
        if rem:
            Gt = n_g - 1
            wait_read(Gt % 2)
            start_adds(Gt, Gt % 2)
        if n_g >= 2:
            wait_adds((n_g - 2) % 2)
        wait_adds((n_g - 1) % 2)
        plsc.subcore_barrier()

        @pl.when(s < NS - 1)
        def _():
            pltpu.sync_copy(acc.at[pl.ds(s * R0, R0)],
                            out_hbm.at[pl.ds(s * R0, R0), pl.ds(c * COLS, COLS)])

        @pl.when(s == NS - 1)
        def _():
            pltpu.sync_copy(acc.at[pl.ds((NS - 1) * R0, LAST)],
                            out_hbm.at[pl.ds((NS - 1) * R0, LAST),
                                       pl.ds(c * COLS, COLS)])

    return sk(hb, dst.reshape(NS, n_ch, CH), zrows)


def kernel(h_atm, edge_index_G, h_bnd, We1, be1, We2, be2, Wn1, bn1, Wn2, bn2):
    N, D = h_atm.shape
    E = h_bnd.shape[0]
    L = We1.shape[0]
    BN = 2000 if N % 2000 == 0 else N
    BE = 4000

    src = edge_index_G[0]
    dst = edge_index_G[1]
    BE = BE if E % BE == 0 else E
    parts = [(0, E)]
    NP = len(parts)

    Wa = We1[:, :D, :]
    Wb = We1[:, D:2 * D, :]
    We = We1[:, 2 * D:, :]
    Wab = jnp.stack([Wa, Wb], axis=1)
    Bab = jnp.stack([be1, jnp.zeros_like(be1)], axis=1)
    Wn1x = Wn1[:, :D, :]
    Wn1a = Wn1[:, D:, :]
    R0 = ((N + NS - 1) // NS + 7) // 8 * 8
    zrows = jnp.zeros((R0, D // NC), dtype=jnp.float32)

    idx_p = [jnp.concatenate([src[o:o + n], dst[o:o + n] + N]) for o, n in parts]
    dst_p = [dst[o:o + n] for o, n in parts]
    hbs = [h_bnd[o:o + n] for o, n in parts]

    T = _make_table(h_atm, Wab[0], Bab[0].reshape(2, 1, D), BN)
    for i in range(L):
        be2_i = be2[i].reshape(1, D)
        Gs = [_gather_rows(T.reshape(2 * N, D // 2), idx_p[p])
              .reshape(2, parts[p][1], D // 2) for p in range(NP)]
        aggs = []
        for p in range(NP):
            hbs[p] = _edge_mlp(Gs[p], hbs[p], We[i], We2[i], be2_i, BE)
            aggs.append(_segment_sum(hbs[p], dst_p[p], zrows, N))
        agg = aggs[0] if NP == 1 else sum(aggs)
        bn1_i = bn1[i].reshape(1, D)
        bn2_i = bn2[i].reshape(1, D)
        if i + 1 < L:
            h_atm, T = _node_mlp_and_table(h_atm, agg, Wn1x[i],
                                           Wn1a[i], bn1_i, Wn2[i], bn2_i,
                                           Wab[i + 1], Bab[i + 1], BN)
        else:
            h_atm = _node_mlp(h_atm, agg, Wn1x[i], Wn1a[i],
                              bn1_i, Wn2[i], bn2_i, BN)
    return (h_atm, hbs[0] if NP == 1 else jnp.concatenate(hbs, axis=0))
